# baseline (speedup 1.0000x reference)
import jax
import jax.numpy as jnp
from jax import lax
from jax.experimental import pallas as pl
from jax.experimental.pallas import tpu as pltpu
from jax.experimental.pallas import tpu_sc as plsc

N = 2048
C = 64
F = 256
NSUB = 16
CPT = C // NSUB
TPT = N // NSUB
NR = 2816
PPT = NR // NSUB
TM = 128


def _route_body(ids_hbm, x_hbm, inv_hbm, off_hbm, nblk_hbm, xr_hbm,
                ids_v, loc_perm, loc_inv, cntg_v, off_v, nblk_v, pcnt_v,
                tmp16_v, permrow_v, tmpp_v, invrow_v, tmpi_v,
                permA_v, permB_v, rowsA_v, rowsB_v,
                cnt_sh, perm_sh, inv_sh, sem):
    sid = lax.axis_index("s")
    lane = lax.iota(jnp.int32, 16)
    c0 = CPT * sid
    zeros16 = jnp.zeros((16,), jnp.int32)

    pltpu.sync_copy(ids_hbm, ids_v)

    def zero_body(i, _):
        loc_perm[pl.ds(i * 16, 16)] = zeros16
        return 0

    lax.fori_loop(0, NR // 16, zero_body, 0)

    def cnt_body(k, acc):
        ids = ids_v[pl.ds(k * 16, 16)]
        return tuple(acc[j] + jnp.where(ids == (c0 + j), 1, 0)
                     for j in range(CPT))

    accs = lax.fori_loop(0, N // 16, cnt_body,
                         tuple(zeros16 for _ in range(CPT)))
    my_cnt = [jnp.sum(accs[j]) for j in range(CPT)]
    row = zeros16
    for j in range(CPT):
        row = jnp.where(lane == j, my_cnt[j], row)
    tmp16_v[...] = row
    pltpu.sync_copy(tmp16_v, cnt_sh.at[pl.ds(sid * 16, 16)])
    plsc.subcore_barrier()

    pltpu.sync_copy(cnt_sh, cntg_v)
    carry = jnp.int32(0)
    for g in range(C // 16):
        flat_idx = ((4 * g + jnp.right_shift(lane, 2)) * 16
                    + jnp.bitwise_and(lane, 3))
        cnt = plsc.load_gather(cntg_v, [flat_idx])
        pcnt = jnp.bitwise_and(cnt + 7, jnp.int32(-8))
        cum = plsc.cumsum(pcnt)
        off = cum - pcnt + carry
        nblk = jnp.right_shift(cnt + (TM - 1), 7)
        off_v[pl.ds(16 * g, 16)] = off
        nblk_v[pl.ds(16 * g, 16)] = nblk
        pcnt_v[pl.ds(16 * g, 16)] = pcnt
        carry = carry + jnp.sum(pcnt)

    @pl.when(sid == 0)
    def _write_meta():
        pltpu.sync_copy(off_v, off_hbm)
        pltpu.sync_copy(nblk_v, nblk_hbm)

    my4 = jnp.bitwise_and(lane, 3)
    myoff = plsc.load_gather(off_v, [c0 + my4])
    mypc = plsc.load_gather(pcnt_v, [c0 + my4])
    bases0 = tuple(jnp.sum(jnp.where(lane == j, myoff, 0)) for j in range(CPT))
    my_pc = [jnp.sum(jnp.where(lane == j, mypc, 0)) for j in range(CPT)]

    def pos_body(k, bases):
        ids = ids_v[pl.ds(k * 16, 16)]
        tok = k * 16 + lane
        val = zeros16
        new_bases = []
        for j in range(CPT):
            m = ids == (c0 + j)
            mc = jnp.where(m, 1, 0)
            incl = plsc.cumsum(mc)
            pos = bases[j] + incl - 1
            val = jnp.where(m, pos, val)
            plsc.store_scatter(loc_perm, [pos], tok, mask=m)
            new_bases.append(bases[j] + jnp.sum(mc))
        loc_inv[pl.ds(k * 16, 16)] = val
        return tuple(new_bases)

    ends = lax.fori_loop(0, N // 16, pos_body, bases0)

    for j in range(CPT):
        npad = (bases0[j] + my_pc[j]) - ends[j]
        plsc.store_scatter(loc_perm, [ends[j] + lane], zeros16,
                           mask=lane < npad)

    pltpu.sync_copy(loc_perm, perm_sh.at[pl.ds(sid * NR, NR)])
    pltpu.sync_copy(loc_inv, inv_sh.at[pl.ds(sid * N, N)])
    plsc.subcore_barrier()
    pbase = sid * PPT
    tbase = sid * TPT
    pltpu.sync_copy(perm_sh.at[pl.ds(pbase, PPT)], permrow_v)
    pltpu.sync_copy(inv_sh.at[pl.ds(tbase, TPT)], invrow_v)
    for r in range(1, NSUB):
        pltpu.sync_copy(perm_sh.at[pl.ds(r * NR + pbase, PPT)], tmpp_v)
        pltpu.sync_copy(inv_sh.at[pl.ds(r * N + tbase, TPT)], tmpi_v)
        for q in range(PPT // 16):
            sl = pl.ds(q * 16, 16)
            permrow_v[sl] = permrow_v[sl] + tmpp_v[sl]
        for q in range(TPT // 16):
            sl = pl.ds(q * 16, 16)
            invrow_v[sl] = invrow_v[sl] + tmpi_v[sl]
    pltpu.sync_copy(invrow_v, inv_hbm.at[pl.ds(tbase, TPT)])

    for q in range(128 // 16):
        sl = pl.ds(q * 16, 16)
        permA_v[sl] = permrow_v[sl]
    for q in range((PPT - 128) // 16):
        permB_v[pl.ds(q * 16, 16)] = permrow_v[pl.ds(128 + q * 16, 16)]
    cp1 = pltpu.async_copy(x_hbm.at[permA_v], rowsA_v, sem)
    cp2 = pltpu.async_copy(x_hbm.at[permB_v], rowsB_v, sem)
    cp1.wait()
    cp2.wait()
    pltpu.sync_copy(rowsA_v, xr_hbm.at[pl.ds(pbase, 128)])
    pltpu.sync_copy(rowsB_v, xr_hbm.at[pl.ds(pbase + 128, PPT - 128)])


def _unroute_body(inv_hbm, yr_hbm, y_hbm, idx_v, rows_v, sem):
    sid = lax.axis_index("s")
    tbase = sid * TPT
    pltpu.sync_copy(inv_hbm.at[pl.ds(tbase, TPT)], idx_v)
    pltpu.async_copy(yr_hbm.at[idx_v], rows_v, sem).wait()
    pltpu.sync_copy(rows_v, y_hbm.at[pl.ds(tbase, TPT)])


def _mm_body(off_ref, nblk_ref, xr_ref, w_ref, b_ref, o_ref):
    c = pl.program_id(0)
    start = pl.multiple_of(off_ref[c], 8)
    n = nblk_ref[c]
    wcat = w_ref[0].astype(jnp.bfloat16)
    brow = b_ref[0]

    def body(i, _):
        rows = xr_ref[pl.ds(start + i * TM, TM), :]
        acc = jnp.dot(rows.astype(jnp.bfloat16), wcat,
                      preferred_element_type=jnp.float32)
        o_ref[pl.ds(start + i * TM, TM), :] = acc + brow
        return 0

    lax.fori_loop(0, n, body, 0)


def _sc_mesh():
    return plsc.VectorSubcoreMesh(core_axis_name="c", subcore_axis_name="s",
                                  num_cores=1)


def kernel(x, cat_ids, W, b):
    ids = cat_ids.astype(jnp.int32)

    route = pl.kernel(
        _route_body,
        out_type=[
            jax.ShapeDtypeStruct((N,), jnp.int32),
            jax.ShapeDtypeStruct((C,), jnp.int32),
            jax.ShapeDtypeStruct((C,), jnp.int32),
            jax.ShapeDtypeStruct((NR, F), jnp.float32),
        ],
        mesh=_sc_mesh(),
        compiler_params=pltpu.CompilerParams(needs_layout_passes=False),
        scratch_types=[
            pltpu.VMEM((N,), jnp.int32),
            pltpu.VMEM((NR,), jnp.int32),
            pltpu.VMEM((N,), jnp.int32),
            pltpu.VMEM((NSUB * 16,), jnp.int32),
            pltpu.VMEM((C,), jnp.int32),
            pltpu.VMEM((C,), jnp.int32),
            pltpu.VMEM((C,), jnp.int32),
            pltpu.VMEM((16,), jnp.int32),
            pltpu.VMEM((PPT,), jnp.int32),
            pltpu.VMEM((PPT,), jnp.int32),
            pltpu.VMEM((TPT,), jnp.int32),
            pltpu.VMEM((TPT,), jnp.int32),
            pltpu.VMEM((128,), jnp.int32),
            pltpu.VMEM((PPT - 128,), jnp.int32),
            pltpu.VMEM((128, F), jnp.float32),
            pltpu.VMEM((PPT - 128, F), jnp.float32),
            pltpu.VMEM_SHARED((NSUB * 16,), jnp.int32),
            pltpu.VMEM_SHARED((NSUB * NR,), jnp.int32),
            pltpu.VMEM_SHARED((NSUB * N,), jnp.int32),
            pltpu.SemaphoreType.DMA,
        ],
    )
    invperm, off, nblk, xr = route(ids, x)

    yr = pl.pallas_call(
        _mm_body,
        grid_spec=pltpu.PrefetchScalarGridSpec(
            num_scalar_prefetch=2,
            grid=(C,),
            in_specs=[
                pl.BlockSpec((NR, F), lambda c, o, nb: (0, 0)),
                pl.BlockSpec((1, F, F), lambda c, o, nb: (c, 0, 0)),
                pl.BlockSpec((1, 1, F), lambda c, o, nb: (c, 0, 0)),
            ],
            out_specs=pl.BlockSpec((NR, F), lambda c, o, nb: (0, 0)),
        ),
        out_shape=jax.ShapeDtypeStruct((NR, F), jnp.float32),
    )(off, nblk, xr, W, b.reshape(C, 1, F))

    unroute = pl.kernel(
        _unroute_body,
        out_type=jax.ShapeDtypeStruct((N, F), jnp.float32),
        mesh=_sc_mesh(),
        compiler_params=pltpu.CompilerParams(needs_layout_passes=False),
        scratch_types=[
            pltpu.VMEM((TPT,), jnp.int32),
            pltpu.VMEM((TPT, F), jnp.float32),
            pltpu.SemaphoreType.DMA,
        ],
    )
    return unroute(invperm, yr)
